# Initial kernel scaffold; baseline (speedup 1.0000x reference)
import functools

import jax
import jax.numpy as jnp
from jax import lax
from jax.experimental import pallas as pl
from jax.experimental.pallas import tpu as pltpu
from jax.experimental.pallas import tpu_sc as plsc

N = 10000
E = 320000
IN_FEAT = 128
OUT_FEAT = 128
NUM_RELS = 8
NUM_BASES = 4

NC = 2
NS = 16
NW = NC * NS
CHUNK = 128
EPW = 10240
EPAD = NW * EPW
NCHUNK = EPW // CHUNK
ROWS_PER_SUB = N // NS


def _basis_body(wc_ref, wv_ref, w3_ref):
    for j in range(NUM_RELS):
        acc = wc_ref[j, 0] * wv_ref[:, 0, :]
        for b in range(1, NUM_BASES):
            acc = acc + wc_ref[j, b] * wv_ref[:, b, :]
        w3_ref[:, j, :] = acc


def _xw_body(x_ref, wflat_ref, xw_ref):
    xb = x_ref[...]
    for r in range(NUM_RELS):
        xw_ref[:, r, :] = jnp.dot(
            xb, wflat_ref[pl.ds(r * IN_FEAT, IN_FEAT), :],
            preferred_element_type=jnp.float32)


def _add_body(a_ref, b_ref, o_ref):
    o_ref[...] = a_ref[...] + b_ref[...]


def _sc_edge_kernel(xw_hbm, src_hbm, dst_hbm, rel_hbm, norm_hbm, out_hbm,
                    srcv, dstv, relv, normv, gidxv, rows, hsh):
    c = lax.axis_index("c")
    s = lax.axis_index("s")
    wid = s * NC + c

    zero16 = jnp.zeros((16,), jnp.float32)
    lanes = lax.iota(jnp.int32, 16)

    def zrow(e, carry):
        eidx = jnp.full((16,), e, jnp.int32)
        for j in range(OUT_FEAT // 16):
            plsc.store_scatter(rows, [eidx, lanes + (j * 16)], zero16)
        return carry

    lax.fori_loop(0, CHUNK, zrow, 0)
    for k in range(ROWS_PER_SUB // 125):
        pltpu.sync_copy(rows.at[pl.ds(0, 125)],
                        hsh.at[pl.ds(s * ROWS_PER_SUB + k * 125, 125)])
    plsc.subcore_barrier()

    base_w = wid * EPW

    def chunk_body(ci, carry):
        base = base_w + ci * CHUNK
        pltpu.sync_copy(src_hbm.at[pl.ds(base, CHUNK)], srcv)
        pltpu.sync_copy(rel_hbm.at[pl.ds(base, CHUNK)], relv)
        pltpu.sync_copy(dst_hbm.at[pl.ds(base, CHUNK)], dstv)
        pltpu.sync_copy(norm_hbm.at[pl.ds(base, CHUNK)], normv)
        for g in range(CHUNK // 16):
            sl = pl.ds(g * 16, 16)
            gidxv[sl] = srcv[sl] * NUM_RELS + relv[sl]
        pltpu.sync_copy(xw_hbm.at[gidxv], rows)

        def mrow(e, carry2):
            eidx = jnp.full((16,), e, jnp.int32)
            nb = plsc.load_gather(normv, [eidx])
            for j in range(OUT_FEAT // 16):
                col = lanes + (j * 16)
                v = plsc.load_gather(rows, [eidx, col])
                plsc.store_scatter(rows, [eidx, col], v * nb)
            return carry2

        lax.fori_loop(0, CHUNK, mrow, 0)
        pltpu.sync_copy(rows, hsh.at[dstv], add=True)
        return carry

    lax.fori_loop(0, NCHUNK, chunk_body, 0)
    plsc.subcore_barrier()
    pltpu.sync_copy(hsh.at[pl.ds(s * ROWS_PER_SUB, ROWS_PER_SUB)],
                    out_hbm.at[c, pl.ds(s * ROWS_PER_SUB, ROWS_PER_SUB)])


def kernel(x, edge_index, rel_type, norm, h_skip, layer_num, weight, w_comp):
    f32 = jnp.float32
    i32 = jnp.int32

    wv = weight.reshape(IN_FEAT, NUM_BASES, OUT_FEAT)
    w3 = pl.pallas_call(
        _basis_body,
        in_specs=[
            pl.BlockSpec(memory_space=pltpu.SMEM),
            pl.BlockSpec((IN_FEAT, NUM_BASES, OUT_FEAT), lambda: (0, 0, 0)),
        ],
        out_specs=pl.BlockSpec((IN_FEAT, NUM_RELS, OUT_FEAT), lambda: (0, 0, 0)),
        out_shape=jax.ShapeDtypeStruct((IN_FEAT, NUM_RELS, OUT_FEAT), f32),
    )(w_comp, wv)
    wflat = w3.reshape(NUM_RELS * IN_FEAT, OUT_FEAT)

    BLK = 1000
    xw = pl.pallas_call(
        _xw_body,
        grid=(N // BLK,),
        in_specs=[
            pl.BlockSpec((BLK, IN_FEAT), lambda i: (i, 0)),
            pl.BlockSpec((NUM_RELS * IN_FEAT, OUT_FEAT), lambda i: (0, 0)),
        ],
        out_specs=pl.BlockSpec((BLK, NUM_RELS, OUT_FEAT), lambda i: (i, 0, 0)),
        out_shape=jax.ShapeDtypeStruct((N, NUM_RELS, OUT_FEAT), f32),
    )(x, wflat)
    xw_flat = xw.reshape(N * NUM_RELS, OUT_FEAT)

    pad = EPAD - E
    src = jnp.concatenate([edge_index[0], jnp.zeros((pad,), i32)])
    dst = jnp.concatenate([edge_index[1], jnp.zeros((pad,), i32)])
    rel = jnp.concatenate([rel_type, jnp.zeros((pad,), i32)])
    nrm = jnp.concatenate([norm[:, 0], jnp.zeros((pad,), f32)])

    mesh = plsc.VectorSubcoreMesh(core_axis_name="c", subcore_axis_name="s")
    hpart = pl.kernel(
        _sc_edge_kernel,
        out_type=jax.ShapeDtypeStruct((NC, N, OUT_FEAT), f32),
        mesh=mesh,
        scratch_types=[
            pltpu.VMEM((CHUNK,), i32),
            pltpu.VMEM((CHUNK,), i32),
            pltpu.VMEM((CHUNK,), i32),
            pltpu.VMEM((CHUNK,), f32),
            pltpu.VMEM((CHUNK,), i32),
            pltpu.VMEM((CHUNK, OUT_FEAT), f32),
            pltpu.VMEM_SHARED((N, OUT_FEAT), f32),
        ],
    )(xw_flat, src, dst, rel, nrm)

    h = pl.pallas_call(
        _add_body,
        grid=(N // BLK,),
        in_specs=[pl.BlockSpec((BLK, OUT_FEAT), lambda i: (i, 0))] * 2,
        out_specs=pl.BlockSpec((BLK, OUT_FEAT), lambda i: (i, 0)),
        out_shape=jax.ShapeDtypeStruct((N, OUT_FEAT), f32),
    )(hpart[0], hpart[1])

    w = wflat.reshape(NUM_RELS, IN_FEAT, OUT_FEAT)
    return (h, w)


# speedup vs baseline: 7.7357x; 7.7357x over previous
import functools

import jax
import jax.numpy as jnp
from jax import lax
from jax.experimental import pallas as pl
from jax.experimental.pallas import tpu as pltpu
from jax.experimental.pallas import tpu_sc as plsc

N = 10000
E = 320000
IN_FEAT = 128
OUT_FEAT = 128
NUM_RELS = 8
NUM_BASES = 4

NC = 2
NS = 16
NW = NC * NS
CHUNK = 128
EPW = 10240
EPAD = NW * EPW
NCHUNK = EPW // CHUNK
NPAD = 10240
ROWS_PER_SUB = NPAD // NS


def _basis_body(wc_ref, wv_ref, w3_ref):
    for j in range(NUM_RELS):
        acc = wc_ref[j, 0] * wv_ref[:, 0, :]
        for b in range(1, NUM_BASES):
            acc = acc + wc_ref[j, b] * wv_ref[:, b, :]
        w3_ref[:, j, :] = acc


def _xw_body(x_ref, wflat_ref, xw_ref):
    xb = x_ref[...]
    for r in range(NUM_RELS):
        xw_ref[:, r, :] = jnp.dot(
            xb, wflat_ref[pl.ds(r * IN_FEAT, IN_FEAT), :],
            preferred_element_type=jnp.float32)


def _add_body(a_ref, b_ref, o_ref):
    o_ref[...] = a_ref[...] + b_ref[...]


def _sc_edge_kernel(xw_hbm, src_hbm, dst_hbm, rel_hbm, norm_hbm, out_hbm,
                    srcv, dstv, relv, normv, gidxv, rows, hsh):
    c = lax.axis_index("c")
    s = lax.axis_index("s")
    wid = s * NC + c

    zero16 = jnp.zeros((16,), jnp.float32)
    lanes = lax.iota(jnp.int32, 16)

    def zrow(e, carry):
        for j in range(OUT_FEAT // 16):
            rows[e, pl.ds(j * 16, 16)] = zero16
        return carry

    lax.fori_loop(0, CHUNK, zrow, 0)
    for k in range(ROWS_PER_SUB // CHUNK):
        pltpu.sync_copy(rows,
                        hsh.at[pl.ds(s * ROWS_PER_SUB + k * CHUNK, CHUNK)])
    plsc.subcore_barrier()

    base_w = wid * EPW

    def chunk_body(ci, carry):
        base = base_w + ci * CHUNK
        pltpu.sync_copy(src_hbm.at[pl.ds(base, CHUNK)], srcv)
        pltpu.sync_copy(rel_hbm.at[pl.ds(base, CHUNK)], relv)
        pltpu.sync_copy(dst_hbm.at[pl.ds(base, CHUNK)], dstv)
        pltpu.sync_copy(norm_hbm.at[pl.ds(base, CHUNK)], normv)
        for g in range(CHUNK // 16):
            sl = pl.ds(g * 16, 16)
            gidxv[sl] = srcv[sl] * NUM_RELS + relv[sl]
        pltpu.sync_copy(xw_hbm.at[gidxv], rows)

        for g in range(CHUNK // 16):
            norm16 = normv[pl.ds(g * 16, 16)]

            def mrow(k, carry2, norm16=norm16, g=g):
                nb = lax.gather(
                    norm16, jnp.full((16, 1), k, jnp.int32),
                    dimension_numbers=lax.GatherDimensionNumbers(
                        offset_dims=(), collapsed_slice_dims=(0,),
                        start_index_map=(0,)),
                    slice_sizes=(1,),
                    mode=lax.GatherScatterMode.PROMISE_IN_BOUNDS)
                e = g * 16 + k
                for j in range(OUT_FEAT // 16):
                    sl = pl.ds(j * 16, 16)
                    rows[e, sl] = rows[e, sl] * nb
                return carry2

            lax.fori_loop(0, 16, mrow, 0)
        pltpu.sync_copy(rows, hsh.at[dstv], add=True)
        return carry

    lax.fori_loop(0, NCHUNK, chunk_body, 0)
    plsc.subcore_barrier()
    pltpu.sync_copy(hsh.at[pl.ds(s * ROWS_PER_SUB, ROWS_PER_SUB)],
                    out_hbm.at[c, pl.ds(s * ROWS_PER_SUB, ROWS_PER_SUB)])


def kernel(x, edge_index, rel_type, norm, h_skip, layer_num, weight, w_comp):
    f32 = jnp.float32
    i32 = jnp.int32

    wv = weight.reshape(IN_FEAT, NUM_BASES, OUT_FEAT)
    w3 = pl.pallas_call(
        _basis_body,
        in_specs=[
            pl.BlockSpec(memory_space=pltpu.SMEM),
            pl.BlockSpec((IN_FEAT, NUM_BASES, OUT_FEAT), lambda: (0, 0, 0)),
        ],
        out_specs=pl.BlockSpec((IN_FEAT, NUM_RELS, OUT_FEAT), lambda: (0, 0, 0)),
        out_shape=jax.ShapeDtypeStruct((IN_FEAT, NUM_RELS, OUT_FEAT), f32),
    )(w_comp, wv)
    wflat = w3.reshape(NUM_RELS * IN_FEAT, OUT_FEAT)

    BLK = 1000
    xw = pl.pallas_call(
        _xw_body,
        grid=(N // BLK,),
        in_specs=[
            pl.BlockSpec((BLK, IN_FEAT), lambda i: (i, 0)),
            pl.BlockSpec((NUM_RELS * IN_FEAT, OUT_FEAT), lambda i: (0, 0)),
        ],
        out_specs=pl.BlockSpec((BLK, NUM_RELS, OUT_FEAT), lambda i: (i, 0, 0)),
        out_shape=jax.ShapeDtypeStruct((N, NUM_RELS, OUT_FEAT), f32),
    )(x, wflat)
    xw_flat = xw.reshape(N * NUM_RELS, OUT_FEAT)

    pad = EPAD - E
    src = jnp.concatenate([edge_index[0], jnp.zeros((pad,), i32)])
    dst = jnp.concatenate([edge_index[1], jnp.zeros((pad,), i32)])
    rel = jnp.concatenate([rel_type, jnp.zeros((pad,), i32)])
    nrm = jnp.concatenate([norm[:, 0], jnp.zeros((pad,), f32)])

    mesh = plsc.VectorSubcoreMesh(core_axis_name="c", subcore_axis_name="s")
    hpart = pl.kernel(
        _sc_edge_kernel,
        out_type=jax.ShapeDtypeStruct((NC, NPAD, OUT_FEAT), f32),
        mesh=mesh,
        scratch_types=[
            pltpu.VMEM((CHUNK,), i32),
            pltpu.VMEM((CHUNK,), i32),
            pltpu.VMEM((CHUNK,), i32),
            pltpu.VMEM((CHUNK,), f32),
            pltpu.VMEM((CHUNK,), i32),
            pltpu.VMEM((CHUNK, OUT_FEAT), f32),
            pltpu.VMEM_SHARED((NPAD, OUT_FEAT), f32),
        ],
    )(xw_flat, src, dst, rel, nrm)

    h = pl.pallas_call(
        _add_body,
        grid=(N // BLK,),
        in_specs=[pl.BlockSpec((BLK, OUT_FEAT), lambda i: (i, 0))] * 2,
        out_specs=pl.BlockSpec((BLK, OUT_FEAT), lambda i: (i, 0)),
        out_shape=jax.ShapeDtypeStruct((N, OUT_FEAT), f32),
    )(hpart[0], hpart[1])

    w = wflat.reshape(NUM_RELS, IN_FEAT, OUT_FEAT)
    return (h, w)


# speedup vs baseline: 13.3570x; 1.7267x over previous
import jax
import jax.numpy as jnp
from jax import lax
from jax.experimental import pallas as pl
from jax.experimental.pallas import tpu as pltpu
from jax.experimental.pallas import tpu_sc as plsc

N = 10000
E = 320000
IN_FEAT = 128
OUT_FEAT = 128
NUM_RELS = 8
NUM_BASES = 4

NC = 2
NS = 16
NW = NC * NS
EPW = 10240
EPAD = NW * EPW
CHUNK = 160
NCHUNK = EPW // CHUNK
NPAD = 10240
ROWS_PER_SUB = NPAD // NS


def _basis_body(wc_ref, wv_ref, w3_ref):
    for j in range(NUM_RELS):
        acc = wc_ref[j, 0] * wv_ref[:, 0, :]
        for b in range(1, NUM_BASES):
            acc = acc + wc_ref[j, b] * wv_ref[:, b, :]
        w3_ref[:, j, :] = acc


def _xw_body(x_ref, wflat_ref, xw_ref):
    xb = x_ref[...]
    for r in range(NUM_RELS):
        xw_ref[:, r, :] = jnp.dot(
            xb, wflat_ref[pl.ds(r * IN_FEAT, IN_FEAT), :],
            preferred_element_type=jnp.float32)


def _gidx_body(src_ref, rel_ref, gidx_ref):
    gidx_ref[...] = src_ref[...] * NUM_RELS + rel_ref[...]


def _add_body(a_ref, b_ref, o_ref):
    o_ref[...] = a_ref[...] + b_ref[...]


def _sc_edge_kernel(xw_hbm, gidx_hbm, dst_hbm, norm_hbm, out_hbm,
                    gidx0, gidx1, dst0, dst1, norm0, norm1, rows0, rows1,
                    sem0, sem1, hsh):
    c = lax.axis_index("c")
    s = lax.axis_index("s")
    wid = s * NC + c

    gidxv = (gidx0, gidx1)
    dstv = (dst0, dst1)
    normv = (norm0, norm1)
    rows = (rows0, rows1)
    sems = (sem0, sem1)

    zero16 = jnp.zeros((16,), jnp.float32)

    def zrow(e, carry):
        for j in range(OUT_FEAT // 16):
            rows0[e, pl.ds(j * 16, 16)] = zero16
        return carry

    lax.fori_loop(0, CHUNK, zrow, 0)
    for k in range(ROWS_PER_SUB // CHUNK):
        pltpu.sync_copy(rows0,
                        hsh.at[pl.ds(s * ROWS_PER_SUB + k * CHUNK, CHUNK)])
    plsc.subcore_barrier()

    base_w = wid * EPW

    def start_gather(ci, b):
        base = base_w + ci * CHUNK
        pltpu.sync_copy(gidx_hbm.at[pl.ds(base, CHUNK)], gidxv[b])
        pltpu.sync_copy(dst_hbm.at[pl.ds(base, CHUNK)], dstv[b])
        pltpu.sync_copy(norm_hbm.at[pl.ds(base, CHUNK)], normv[b])
        pltpu.async_copy(xw_hbm.at[gidxv[b]], rows[b], sems[b])

    def finish_chunk(ci, b):
        pltpu.make_async_copy(
            xw_hbm.at[pl.ds(0, CHUNK)], rows[b], sems[b]).wait()
        for g in range(CHUNK // 16):
            norm16 = normv[b][pl.ds(g * 16, 16)]

            def mrow(k, carry2, norm16=norm16, g=g, b=b):
                nb = lax.gather(
                    norm16, jnp.full((16, 1), k, jnp.int32),
                    dimension_numbers=lax.GatherDimensionNumbers(
                        offset_dims=(), collapsed_slice_dims=(0,),
                        start_index_map=(0,)),
                    slice_sizes=(1,),
                    mode=lax.GatherScatterMode.PROMISE_IN_BOUNDS)
                e = g * 16 + k
                for j in range(OUT_FEAT // 16):
                    sl = pl.ds(j * 16, 16)
                    rows[b][e, sl] = rows[b][e, sl] * nb
                return carry2

            lax.fori_loop(0, 16, mrow, 0)
        pltpu.sync_copy(rows[b], hsh.at[dstv[b]], add=True)

    start_gather(0, 0)

    def pair_body(p, carry):
        ci = 2 * p
        start_gather(ci + 1, 1)
        finish_chunk(ci, 0)
        start_gather(ci + 2, 0)
        finish_chunk(ci + 1, 1)
        return carry

    lax.fori_loop(0, NCHUNK // 2 - 1, pair_body, 0)
    start_gather(NCHUNK - 1, 1)
    finish_chunk(NCHUNK - 2, 0)
    finish_chunk(NCHUNK - 1, 1)

    plsc.subcore_barrier()
    pltpu.sync_copy(hsh.at[pl.ds(s * ROWS_PER_SUB, ROWS_PER_SUB)],
                    out_hbm.at[c, pl.ds(s * ROWS_PER_SUB, ROWS_PER_SUB)])


def kernel(x, edge_index, rel_type, norm, h_skip, layer_num, weight, w_comp):
    f32 = jnp.float32

    wv = weight.reshape(IN_FEAT, NUM_BASES, OUT_FEAT)
    w3 = pl.pallas_call(
        _basis_body,
        in_specs=[
            pl.BlockSpec(memory_space=pltpu.SMEM),
            pl.BlockSpec((IN_FEAT, NUM_BASES, OUT_FEAT), lambda: (0, 0, 0)),
        ],
        out_specs=pl.BlockSpec((IN_FEAT, NUM_RELS, OUT_FEAT), lambda: (0, 0, 0)),
        out_shape=jax.ShapeDtypeStruct((IN_FEAT, NUM_RELS, OUT_FEAT), f32),
    )(w_comp, wv)
    wflat = w3.reshape(NUM_RELS * IN_FEAT, OUT_FEAT)

    BLK = 1000
    xw = pl.pallas_call(
        _xw_body,
        grid=(N // BLK,),
        in_specs=[
            pl.BlockSpec((BLK, IN_FEAT), lambda i: (i, 0)),
            pl.BlockSpec((NUM_RELS * IN_FEAT, OUT_FEAT), lambda i: (0, 0)),
        ],
        out_specs=pl.BlockSpec((BLK, NUM_RELS, OUT_FEAT), lambda i: (i, 0, 0)),
        out_shape=jax.ShapeDtypeStruct((N, NUM_RELS, OUT_FEAT), f32),
    )(x, wflat)
    xw_flat = xw.reshape(N * NUM_RELS, OUT_FEAT)

    EB = 2500
    gidx = pl.pallas_call(
        _gidx_body,
        in_specs=[pl.BlockSpec((EB, IN_FEAT), lambda: (0, 0))] * 2,
        out_specs=pl.BlockSpec((EB, IN_FEAT), lambda: (0, 0)),
        out_shape=jax.ShapeDtypeStruct((EB, IN_FEAT), jnp.int32),
    )(edge_index[0].reshape(EB, IN_FEAT), rel_type.reshape(EB, IN_FEAT))

    pad = EPAD - E
    i32z = jnp.zeros((pad,), jnp.int32)
    gidx1 = jnp.concatenate([gidx.reshape(E), i32z])
    dst1 = jnp.concatenate([edge_index[1], i32z])
    nrm = jnp.concatenate([norm.reshape(E), jnp.zeros((pad,), f32)])

    mesh = plsc.VectorSubcoreMesh(core_axis_name="c", subcore_axis_name="s")
    hpart = pl.kernel(
        _sc_edge_kernel,
        out_type=jax.ShapeDtypeStruct((NC, NPAD, OUT_FEAT), f32),
        mesh=mesh,
        scratch_types=[
            pltpu.VMEM((CHUNK,), jnp.int32),
            pltpu.VMEM((CHUNK,), jnp.int32),
            pltpu.VMEM((CHUNK,), jnp.int32),
            pltpu.VMEM((CHUNK,), jnp.int32),
            pltpu.VMEM((CHUNK,), f32),
            pltpu.VMEM((CHUNK,), f32),
            pltpu.VMEM((CHUNK, OUT_FEAT), f32),
            pltpu.VMEM((CHUNK, OUT_FEAT), f32),
            pltpu.SemaphoreType.DMA,
            pltpu.SemaphoreType.DMA,
            pltpu.VMEM_SHARED((NPAD, OUT_FEAT), f32),
        ],
    )(xw_flat, gidx1, dst1, nrm)

    h = pl.pallas_call(
        _add_body,
        grid=(N // BLK,),
        in_specs=[pl.BlockSpec((BLK, OUT_FEAT), lambda i: (i, 0))] * 2,
        out_specs=pl.BlockSpec((BLK, OUT_FEAT), lambda i: (i, 0)),
        out_shape=jax.ShapeDtypeStruct((N, OUT_FEAT), f32),
    )(hpart[0, :N], hpart[1, :N])

    w = wflat.reshape(NUM_RELS, IN_FEAT, OUT_FEAT)
    return (h, w)


# speedup vs baseline: 21.7609x; 1.6292x over previous
import jax
import jax.numpy as jnp
from jax import lax
from jax.experimental import pallas as pl
from jax.experimental.pallas import tpu as pltpu
from jax.experimental.pallas import tpu_sc as plsc

N = 10000
E = 320000
IN_FEAT = 128
OUT_FEAT = 128
NUM_RELS = 8
NUM_BASES = 4

NC = 2
NS = 16
NW = NC * NS
EPW = 10240
EPAD = NW * EPW
CHUNK = 160
NCHUNK = EPW // CHUNK
NPAD = 10240
ROWS_PER_SUB = NPAD // NS


def _basis_body(wc_ref, wv_ref, w3_ref):
    for j in range(NUM_RELS):
        acc = wc_ref[j, 0] * wv_ref[:, 0, :]
        for b in range(1, NUM_BASES):
            acc = acc + wc_ref[j, b] * wv_ref[:, b, :]
        w3_ref[:, j, :] = acc


def _xw_body(x_ref, wflat_ref, xw_ref):
    xb = x_ref[...]
    for r in range(NUM_RELS):
        xw_ref[:, r, :] = jnp.dot(
            xb, wflat_ref[pl.ds(r * IN_FEAT, IN_FEAT), :],
            preferred_element_type=jnp.float32)


def _gidx_body(src_ref, rel_ref, gidx_ref):
    gidx_ref[...] = src_ref[...] * NUM_RELS + rel_ref[...]


def _add_body(a_ref, b_ref, o_ref):
    o_ref[...] = a_ref[...] + b_ref[...]


def _sc_edge_kernel(xw_hbm, gidx_hbm, dst_hbm, norm_hbm, out_hbm,
                    gidx0, gidx1, dst0, dst1, norm0, norm1, rows0, rows1,
                    sem0, sem1, hsh):
    c = lax.axis_index("c")
    s = lax.axis_index("s")
    wid = s * NC + c

    gidxv = (gidx0, gidx1)
    dstv = (dst0, dst1)
    normv = (norm0, norm1)
    rows = (rows0, rows1)
    sems = (sem0, sem1)

    zero16 = jnp.zeros((16,), jnp.float32)

    def zrow(e, carry):
        for j in range(OUT_FEAT // 16):
            rows0[e, pl.ds(j * 16, 16)] = zero16
        return carry

    lax.fori_loop(0, CHUNK, zrow, 0)
    for k in range(ROWS_PER_SUB // CHUNK):
        pltpu.sync_copy(rows0,
                        hsh.at[pl.ds(s * ROWS_PER_SUB + k * CHUNK, CHUNK)])
    plsc.subcore_barrier()

    base_w = wid * EPW

    def start_gather(ci, b):
        base = base_w + ci * CHUNK
        pltpu.sync_copy(gidx_hbm.at[pl.ds(base, CHUNK)], gidxv[b])
        pltpu.sync_copy(dst_hbm.at[pl.ds(base, CHUNK)], dstv[b])
        pltpu.sync_copy(norm_hbm.at[pl.ds(base, CHUNK)], normv[b])
        pltpu.async_copy(xw_hbm.at[gidxv[b]], rows[b], sems[b])

    def finish_chunk(ci, b):
        pltpu.make_async_copy(
            xw_hbm.at[pl.ds(0, CHUNK)], rows[b], sems[b]).wait()
        for g in range(CHUNK // 16):
            norm16 = normv[b][pl.ds(g * 16, 16)]

            def mrow(k, carry2, norm16=norm16, g=g, b=b):
                nb = lax.gather(
                    norm16, jnp.full((16, 1), k, jnp.int32),
                    dimension_numbers=lax.GatherDimensionNumbers(
                        offset_dims=(), collapsed_slice_dims=(0,),
                        start_index_map=(0,)),
                    slice_sizes=(1,),
                    mode=lax.GatherScatterMode.PROMISE_IN_BOUNDS)
                e = g * 16 + k
                for j in range(OUT_FEAT // 16):
                    sl = pl.ds(j * 16, 16)
                    rows[b][e, sl] = rows[b][e, sl] * nb
                return carry2

            lax.fori_loop(0, 16, mrow, 0)
        pltpu.sync_copy(rows[b], hsh.at[dstv[b]], add=True)

    start_gather(0, 0)

    def pair_body(p, carry):
        ci = 2 * p
        start_gather(ci + 1, 1)
        finish_chunk(ci, 0)
        start_gather(ci + 2, 0)
        finish_chunk(ci + 1, 1)
        return carry

    lax.fori_loop(0, NCHUNK // 2 - 1, pair_body, 0)
    start_gather(NCHUNK - 1, 1)
    finish_chunk(NCHUNK - 2, 0)
    finish_chunk(NCHUNK - 1, 1)

    plsc.subcore_barrier()
    pltpu.sync_copy(hsh.at[pl.ds(s * ROWS_PER_SUB, ROWS_PER_SUB)],
                    out_hbm.at[c, pl.ds(s * ROWS_PER_SUB, ROWS_PER_SUB)])


def kernel(x, edge_index, rel_type, norm, h_skip, layer_num, weight, w_comp):
    f32 = jnp.float32

    wv = weight.reshape(IN_FEAT, NUM_BASES, OUT_FEAT)
    w3 = pl.pallas_call(
        _basis_body,
        in_specs=[
            pl.BlockSpec(memory_space=pltpu.SMEM),
            pl.BlockSpec((IN_FEAT, NUM_BASES, OUT_FEAT), lambda: (0, 0, 0)),
        ],
        out_specs=pl.BlockSpec((IN_FEAT, NUM_RELS, OUT_FEAT), lambda: (0, 0, 0)),
        out_shape=jax.ShapeDtypeStruct((IN_FEAT, NUM_RELS, OUT_FEAT), f32),
    )(w_comp, wv)
    wflat = w3.reshape(NUM_RELS * IN_FEAT, OUT_FEAT)

    BLK = 1000
    xw = pl.pallas_call(
        _xw_body,
        grid=(N // BLK,),
        in_specs=[
            pl.BlockSpec((BLK, IN_FEAT), lambda i: (i, 0)),
            pl.BlockSpec((NUM_RELS * IN_FEAT, OUT_FEAT), lambda i: (0, 0)),
        ],
        out_specs=pl.BlockSpec((BLK, NUM_RELS, OUT_FEAT), lambda i: (i, 0, 0)),
        out_shape=jax.ShapeDtypeStruct((N, NUM_RELS, OUT_FEAT), f32),
    )(x, wflat)
    xw_flat = xw.reshape(N * NUM_RELS, OUT_FEAT)

    EB = 2500
    gidx = pl.pallas_call(
        _gidx_body,
        in_specs=[pl.BlockSpec((EB, IN_FEAT), lambda: (0, 0))] * 2,
        out_specs=pl.BlockSpec((EB, IN_FEAT), lambda: (0, 0)),
        out_shape=jax.ShapeDtypeStruct((EB, IN_FEAT), jnp.int32),
    )(edge_index[0].reshape(EB, IN_FEAT), rel_type.reshape(EB, IN_FEAT))

    epw_real = E // NW
    pad_w = EPW - epw_real
    pad_gidx = jnp.broadcast_to(
        (jnp.arange(pad_w, dtype=jnp.int32) * NUM_RELS)[None], (NW, pad_w))
    pad_dst = jnp.broadcast_to(
        (N + jnp.arange(pad_w, dtype=jnp.int32))[None], (NW, pad_w))
    gidx1 = jnp.concatenate(
        [gidx.reshape(NW, epw_real), pad_gidx], axis=1).reshape(EPAD)
    dst1 = jnp.concatenate(
        [edge_index[1].reshape(NW, epw_real), pad_dst], axis=1).reshape(EPAD)
    nrm = jnp.concatenate(
        [norm.reshape(NW, epw_real), jnp.zeros((NW, pad_w), f32)],
        axis=1).reshape(EPAD)

    mesh = plsc.VectorSubcoreMesh(core_axis_name="c", subcore_axis_name="s")
    hpart = pl.kernel(
        _sc_edge_kernel,
        out_type=jax.ShapeDtypeStruct((NC, NPAD, OUT_FEAT), f32),
        mesh=mesh,
        scratch_types=[
            pltpu.VMEM((CHUNK,), jnp.int32),
            pltpu.VMEM((CHUNK,), jnp.int32),
            pltpu.VMEM((CHUNK,), jnp.int32),
            pltpu.VMEM((CHUNK,), jnp.int32),
            pltpu.VMEM((CHUNK,), f32),
            pltpu.VMEM((CHUNK,), f32),
            pltpu.VMEM((CHUNK, OUT_FEAT), f32),
            pltpu.VMEM((CHUNK, OUT_FEAT), f32),
            pltpu.SemaphoreType.DMA,
            pltpu.SemaphoreType.DMA,
            pltpu.VMEM_SHARED((NPAD, OUT_FEAT), f32),
        ],
    )(xw_flat, gidx1, dst1, nrm)

    h = pl.pallas_call(
        _add_body,
        grid=(N // BLK,),
        in_specs=[pl.BlockSpec((BLK, OUT_FEAT), lambda i: (i, 0))] * 2,
        out_specs=pl.BlockSpec((BLK, OUT_FEAT), lambda i: (i, 0)),
        out_shape=jax.ShapeDtypeStruct((N, OUT_FEAT), f32),
    )(hpart[0, :N], hpart[1, :N])

    w = wflat.reshape(NUM_RELS, IN_FEAT, OUT_FEAT)
    return (h, w)


# speedup vs baseline: 22.2825x; 1.0240x over previous
import jax
import jax.numpy as jnp
from jax import lax
from jax.experimental import pallas as pl
from jax.experimental.pallas import tpu as pltpu
from jax.experimental.pallas import tpu_sc as plsc

N = 10000
E = 320000
IN_FEAT = 128
OUT_FEAT = 128
NUM_RELS = 8
NUM_BASES = 4

NC = 2
NS = 16
NW = NC * NS
EPW = 10240
EPAD = NW * EPW
CHUNK = 160
NCHUNK = EPW // CHUNK
NPAD = 10240
ROWS_PER_SUB = NPAD // NS


def _basis_body(wc_ref, wv_ref, w3_ref):
    for j in range(NUM_RELS):
        acc = wc_ref[j, 0] * wv_ref[:, 0, :]
        for b in range(1, NUM_BASES):
            acc = acc + wc_ref[j, b] * wv_ref[:, b, :]
        w3_ref[:, j, :] = acc


def _xw_body(x_ref, wflat_ref, xw_ref):
    xb = x_ref[...]
    for r in range(NUM_RELS):
        xw_ref[:, r, :] = jnp.dot(
            xb, wflat_ref[pl.ds(r * IN_FEAT, IN_FEAT), :],
            preferred_element_type=jnp.float32)


def _gidx_body(src_ref, rel_ref, gidx_ref):
    gidx_ref[...] = src_ref[...] * NUM_RELS + rel_ref[...]


def _add_body(a_ref, b_ref, o_ref):
    o_ref[...] = a_ref[0] + b_ref[0]


def _sc_edge_kernel(xw_hbm, gidx_hbm, dst_hbm, norm_hbm, out_hbm,
                    gidx0, gidx1, dst0, dst1, norm0, norm1, rows0, rows1,
                    sem0, sem1, hsh):
    c = lax.axis_index("c")
    s = lax.axis_index("s")
    wid = s * NC + c

    gidxv = (gidx0, gidx1)
    dstv = (dst0, dst1)
    normv = (norm0, norm1)
    rows = (rows0, rows1)
    sems = (sem0, sem1)

    zero16 = jnp.zeros((16,), jnp.float32)

    def zrow(e, carry):
        for j in range(OUT_FEAT // 16):
            rows0[e, pl.ds(j * 16, 16)] = zero16
        return carry

    ZB = 128
    lax.fori_loop(0, ZB, zrow, 0)
    for k in range(ROWS_PER_SUB // ZB):
        pltpu.sync_copy(rows0.at[pl.ds(0, ZB)],
                        hsh.at[pl.ds(s * ROWS_PER_SUB + k * ZB, ZB)])
    plsc.subcore_barrier()

    base_w = wid * EPW

    def start_gather(ci, b):
        base = base_w + ci * CHUNK
        pltpu.sync_copy(gidx_hbm.at[pl.ds(base, CHUNK)], gidxv[b])
        pltpu.sync_copy(dst_hbm.at[pl.ds(base, CHUNK)], dstv[b])
        pltpu.sync_copy(norm_hbm.at[pl.ds(base, CHUNK)], normv[b])
        pltpu.async_copy(xw_hbm.at[gidxv[b]], rows[b], sems[b])

    def finish_chunk(ci, b):
        pltpu.make_async_copy(
            xw_hbm.at[pl.ds(0, CHUNK)], rows[b], sems[b]).wait()
        for g in range(CHUNK // 16):
            norm16 = normv[b][pl.ds(g * 16, 16)]

            def mrow(k, carry2, norm16=norm16, g=g, b=b):
                nb = lax.gather(
                    norm16, jnp.full((16, 1), k, jnp.int32),
                    dimension_numbers=lax.GatherDimensionNumbers(
                        offset_dims=(), collapsed_slice_dims=(0,),
                        start_index_map=(0,)),
                    slice_sizes=(1,),
                    mode=lax.GatherScatterMode.PROMISE_IN_BOUNDS)
                e = g * 16 + k
                for j in range(OUT_FEAT // 16):
                    sl = pl.ds(j * 16, 16)
                    rows[b][e, sl] = rows[b][e, sl] * nb
                return carry2

            lax.fori_loop(0, 16, mrow, 0)
        pltpu.sync_copy(rows[b], hsh.at[dstv[b]], add=True)

    start_gather(0, 0)

    def pair_body(p, carry):
        ci = 2 * p
        start_gather(ci + 1, 1)
        finish_chunk(ci, 0)
        start_gather(ci + 2, 0)
        finish_chunk(ci + 1, 1)
        return carry

    lax.fori_loop(0, NCHUNK // 2 - 1, pair_body, 0)
    start_gather(NCHUNK - 1, 1)
    finish_chunk(NCHUNK - 2, 0)
    finish_chunk(NCHUNK - 1, 1)

    plsc.subcore_barrier()
    pltpu.sync_copy(hsh.at[pl.ds(s * ROWS_PER_SUB, ROWS_PER_SUB)],
                    out_hbm.at[c, pl.ds(s * ROWS_PER_SUB, ROWS_PER_SUB)])


def kernel(x, edge_index, rel_type, norm, h_skip, layer_num, weight, w_comp):
    f32 = jnp.float32

    wv = weight.reshape(IN_FEAT, NUM_BASES, OUT_FEAT)
    w3 = pl.pallas_call(
        _basis_body,
        in_specs=[
            pl.BlockSpec(memory_space=pltpu.SMEM),
            pl.BlockSpec((IN_FEAT, NUM_BASES, OUT_FEAT), lambda: (0, 0, 0)),
        ],
        out_specs=pl.BlockSpec((IN_FEAT, NUM_RELS, OUT_FEAT), lambda: (0, 0, 0)),
        out_shape=jax.ShapeDtypeStruct((IN_FEAT, NUM_RELS, OUT_FEAT), f32),
    )(w_comp, wv)
    wflat = w3.reshape(NUM_RELS * IN_FEAT, OUT_FEAT)

    BLK = 1000
    xw = pl.pallas_call(
        _xw_body,
        grid=(N // BLK,),
        in_specs=[
            pl.BlockSpec((BLK, IN_FEAT), lambda i: (i, 0)),
            pl.BlockSpec((NUM_RELS * IN_FEAT, OUT_FEAT), lambda i: (0, 0)),
        ],
        out_specs=pl.BlockSpec((BLK, NUM_RELS, OUT_FEAT), lambda i: (i, 0, 0)),
        out_shape=jax.ShapeDtypeStruct((N, NUM_RELS, OUT_FEAT), f32),
    )(x, wflat)
    xw_flat = xw.reshape(N * NUM_RELS, OUT_FEAT)

    EB = 2500
    gidx = pl.pallas_call(
        _gidx_body,
        in_specs=[pl.BlockSpec((EB, IN_FEAT), lambda: (0, 0))] * 2,
        out_specs=pl.BlockSpec((EB, IN_FEAT), lambda: (0, 0)),
        out_shape=jax.ShapeDtypeStruct((EB, IN_FEAT), jnp.int32),
    )(edge_index[0].reshape(EB, IN_FEAT), rel_type.reshape(EB, IN_FEAT))

    epw_real = E // NW
    pad_w = EPW - epw_real
    pad_gidx = jnp.broadcast_to(
        (jnp.arange(pad_w, dtype=jnp.int32) * NUM_RELS)[None], (NW, pad_w))
    pad_dst = jnp.broadcast_to(
        (N + jnp.arange(pad_w, dtype=jnp.int32))[None], (NW, pad_w))
    gidx1 = jnp.concatenate(
        [gidx.reshape(NW, epw_real), pad_gidx], axis=1).reshape(EPAD)
    dst1 = jnp.concatenate(
        [edge_index[1].reshape(NW, epw_real), pad_dst], axis=1).reshape(EPAD)
    nrm = jnp.concatenate(
        [norm.reshape(NW, epw_real), jnp.zeros((NW, pad_w), f32)],
        axis=1).reshape(EPAD)

    mesh = plsc.VectorSubcoreMesh(core_axis_name="c", subcore_axis_name="s")
    hpart = pl.kernel(
        _sc_edge_kernel,
        out_type=jax.ShapeDtypeStruct((NC, NPAD, OUT_FEAT), f32),
        mesh=mesh,
        scratch_types=[
            pltpu.VMEM((CHUNK,), jnp.int32),
            pltpu.VMEM((CHUNK,), jnp.int32),
            pltpu.VMEM((CHUNK,), jnp.int32),
            pltpu.VMEM((CHUNK,), jnp.int32),
            pltpu.VMEM((CHUNK,), f32),
            pltpu.VMEM((CHUNK,), f32),
            pltpu.VMEM((CHUNK, OUT_FEAT), f32),
            pltpu.VMEM((CHUNK, OUT_FEAT), f32),
            pltpu.SemaphoreType.DMA,
            pltpu.SemaphoreType.DMA,
            pltpu.VMEM_SHARED((NPAD, OUT_FEAT), f32),
        ],
    )(xw_flat, gidx1, dst1, nrm)

    h = pl.pallas_call(
        _add_body,
        grid=(N // BLK,),
        in_specs=[
            pl.BlockSpec((1, BLK, OUT_FEAT), lambda i: (0, i, 0)),
            pl.BlockSpec((1, BLK, OUT_FEAT), lambda i: (1, i, 0)),
        ],
        out_specs=pl.BlockSpec((BLK, OUT_FEAT), lambda i: (i, 0)),
        out_shape=jax.ShapeDtypeStruct((N, OUT_FEAT), f32),
    )(hpart, hpart)

    w = wflat.reshape(NUM_RELS, IN_FEAT, OUT_FEAT)
    return (h, w)
